# baseline (speedup 1.0000x reference)
import math

import jax
import jax.numpy as jnp
from jax.experimental import pallas as pl
from jax.experimental.pallas import tpu as pltpu

_B, _S, _DIN, _DOUT = 32, 64, 2048, 2048
_BASE_LR = 0.01
_SCALAR_SCALER = math.sqrt(_DIN)
_LR_SCALE = _BASE_LR * math.sqrt(max(_DIN, _DOUT)) * math.sqrt(1.0 / _DIN)

_BO = 1024


def _body(x_ref, log_lr_hbm, state_ref, base_hbm, o_ref, lr_vmem, base_vmem,
          sems):
    ob = pl.program_id(0)
    b = pl.program_id(1)

    @pl.when(jnp.logical_and(b == 0, ob == 0))
    def _():
        lr_cp = pltpu.make_async_copy(log_lr_hbm, lr_vmem, sems.at[0])
        base_cp = pltpu.make_async_copy(base_hbm, base_vmem, sems.at[1])
        lr_cp.start()
        base_cp.start()
        lr_cp.wait()
        base_cp.wait()
        lr_vmem[...] = _LR_SCALE * jnp.exp(lr_vmem[...] * _SCALAR_SCALER)

    rows = pl.ds(ob * _BO, _BO)
    w = lr_vmem[rows, :] * state_ref[0] + base_vmem[rows, :]
    o_ref[0] = jax.lax.dot_general(
        x_ref[0], w, (((1,), (1,)), ((), ())),
        preferred_element_type=jnp.float32)


def _call(x, log_lr, state, base_w, interpret=False):
    n_ob = _DOUT // _BO
    return pl.pallas_call(
        _body,
        out_shape=jax.ShapeDtypeStruct((_B, _S, _DOUT), jnp.float32),
        grid=(n_ob, _B),
        in_specs=[
            pl.BlockSpec((1, _S, _DIN), lambda ob, b: (b, 0, 0)),
            pl.BlockSpec(memory_space=pl.ANY),
            pl.BlockSpec((1, _BO, _DIN), lambda ob, b: (b, ob, 0)),
            pl.BlockSpec(memory_space=pl.ANY),
        ],
        out_specs=pl.BlockSpec((1, _S, _BO), lambda ob, b: (b, 0, ob)),
        scratch_shapes=[
            pltpu.VMEM((_DOUT, _DIN), jnp.float32),
            pltpu.VMEM((_DOUT, _DIN), jnp.float32),
            pltpu.SemaphoreType.DMA((2,)),
        ],
        compiler_params=pltpu.CompilerParams(
            dimension_semantics=("parallel", "arbitrary"),
            vmem_limit_bytes=56 * 1024 * 1024,
        ),
        name="ittt_linear",
        interpret=interpret,
    )(x, log_lr, state, base_w)


def kernel(x, log_lr, state, momentum, base_w):
    del momentum
    return _call(x, log_lr, state, base_w)


# speedup vs baseline: 1.0909x; 1.0909x over previous
import math

import jax
import jax.numpy as jnp
from jax.experimental import pallas as pl
from jax.experimental.pallas import tpu as pltpu

_B, _S, _DIN, _DOUT = 32, 64, 2048, 2048
_BASE_LR = 0.01
_SCALAR_SCALER = math.sqrt(_DIN)
_LR_SCALE = _BASE_LR * math.sqrt(max(_DIN, _DOUT)) * math.sqrt(1.0 / _DIN)

_BO = 1024
_STAGE = 512


def _body(x_ref, log_lr_hbm, state_ref, base_hbm, o_ref, lr_bf, base_bf, sem):
    b = pl.program_id(0)
    ob = pl.program_id(1)

    @pl.when(jnp.logical_and(b == 0, ob == 0))
    def _():
        def stage(tmp_ref):
            for i in range(_DOUT // _STAGE):
                rows = slice(i * _STAGE, (i + 1) * _STAGE)
                cp = pltpu.make_async_copy(log_lr_hbm.at[rows, :], tmp_ref, sem)
                cp.start()
                cp.wait()
                lr_bf[rows, :] = (
                    _LR_SCALE * jnp.exp(tmp_ref[...] * _SCALAR_SCALER)
                ).astype(jnp.bfloat16)
            for i in range(_DOUT // _STAGE):
                rows = slice(i * _STAGE, (i + 1) * _STAGE)
                cp = pltpu.make_async_copy(base_hbm.at[rows, :], tmp_ref, sem)
                cp.start()
                cp.wait()
                base_bf[rows, :] = tmp_ref[...].astype(jnp.bfloat16)

        pl.run_scoped(stage, pltpu.VMEM((_STAGE, _DIN), jnp.float32))

    rows = pl.ds(ob * _BO, _BO)
    w = lr_bf[rows, :] * state_ref[0].astype(jnp.bfloat16) + base_bf[rows, :]
    o_ref[0] = jax.lax.dot_general(
        x_ref[0].astype(jnp.bfloat16), w, (((1,), (1,)), ((), ())),
        preferred_element_type=jnp.float32)


def _call(x, log_lr, state, base_w, interpret=False):
    n_ob = _DOUT // _BO
    return pl.pallas_call(
        _body,
        out_shape=jax.ShapeDtypeStruct((_B, _S, _DOUT), jnp.float32),
        grid=(_B, n_ob),
        in_specs=[
            pl.BlockSpec((1, _S, _DIN), lambda b, ob: (b, 0, 0)),
            pl.BlockSpec(memory_space=pl.ANY),
            pl.BlockSpec((1, _BO, _DIN), lambda b, ob: (b, ob, 0)),
            pl.BlockSpec(memory_space=pl.ANY),
        ],
        out_specs=pl.BlockSpec((1, _S, _BO), lambda b, ob: (b, 0, ob)),
        scratch_shapes=[
            pltpu.VMEM((_DOUT, _DIN), jnp.bfloat16),
            pltpu.VMEM((_DOUT, _DIN), jnp.bfloat16),
            pltpu.SemaphoreType.DMA,
        ],
        compiler_params=pltpu.CompilerParams(
            dimension_semantics=("parallel", "arbitrary"),
            vmem_limit_bytes=56 * 1024 * 1024,
        ),
        name="ittt_linear",
        interpret=interpret,
    )(x, log_lr, state, base_w)


def kernel(x, log_lr, state, momentum, base_w):
    del momentum
    return _call(x, log_lr, state, base_w)


# speedup vs baseline: 1.1336x; 1.0391x over previous
import math

import jax
import jax.numpy as jnp
from jax.experimental import pallas as pl
from jax.experimental.pallas import tpu as pltpu

_B, _S, _DIN, _DOUT = 32, 64, 2048, 2048
_BASE_LR = 0.01
_SCALAR_SCALER = math.sqrt(_DIN)
_LR_SCALE = _BASE_LR * math.sqrt(max(_DIN, _DOUT)) * math.sqrt(1.0 / _DIN)

_BO = 1024


def _body(x_ref, log_lr_ref, state_ref, base_ref, o_ref, lr_scr, base_scr):
    b = pl.program_id(1)

    @pl.when(b == 0)
    def _():
        lr_scr[...] = (
            _LR_SCALE * jnp.exp(log_lr_ref[...] * _SCALAR_SCALER)
        ).astype(jnp.bfloat16)
        base_scr[...] = base_ref[...].astype(jnp.bfloat16)

    w = lr_scr[...] * state_ref[0].astype(jnp.bfloat16) + base_scr[...]
    o_ref[0] = jax.lax.dot_general(
        x_ref[0].astype(jnp.bfloat16), w, (((1,), (1,)), ((), ())),
        preferred_element_type=jnp.float32)


def _call(x, log_lr, state, base_w, interpret=False):
    n_ob = _DOUT // _BO
    return pl.pallas_call(
        _body,
        out_shape=jax.ShapeDtypeStruct((_B, _S, _DOUT), jnp.float32),
        grid=(n_ob, _B),
        in_specs=[
            pl.BlockSpec((1, _S, _DIN), lambda ob, b: (b, 0, 0)),
            pl.BlockSpec((_BO, _DIN), lambda ob, b: (ob, 0)),
            pl.BlockSpec((1, _BO, _DIN), lambda ob, b: (b, ob, 0)),
            pl.BlockSpec((_BO, _DIN), lambda ob, b: (ob, 0)),
        ],
        out_specs=pl.BlockSpec((1, _S, _BO), lambda ob, b: (b, 0, ob)),
        scratch_shapes=[
            pltpu.VMEM((_BO, _DIN), jnp.bfloat16),
            pltpu.VMEM((_BO, _DIN), jnp.bfloat16),
        ],
        compiler_params=pltpu.CompilerParams(
            dimension_semantics=("parallel", "arbitrary"),
            vmem_limit_bytes=58 * 1024 * 1024,
        ),
        name="ittt_linear",
        interpret=interpret,
    )(x, log_lr, state, base_w)


def kernel(x, log_lr, state, momentum, base_w):
    del momentum
    return _call(x, log_lr, state, base_w)


# speedup vs baseline: 1.1423x; 1.0076x over previous
import math

import jax
import jax.numpy as jnp
from jax.experimental import pallas as pl
from jax.experimental.pallas import tpu as pltpu

_B, _S, _DIN, _DOUT = 32, 64, 2048, 2048
_BASE_LR = 0.01
_SCALAR_SCALER = math.sqrt(_DIN)
_LR_SCALE = _BASE_LR * math.sqrt(max(_DIN, _DOUT)) * math.sqrt(1.0 / _DIN)

_BO = 1024


def _body(x_ref, log_lr_ref, state_ref, base_ref, o_ref, lr_scr):
    b = pl.program_id(1)

    @pl.when(b == 0)
    def _():
        lr_scr[...] = (
            _LR_SCALE * jnp.exp(log_lr_ref[...] * _SCALAR_SCALER)
        ).astype(jnp.bfloat16)

    w = lr_scr[...].astype(jnp.float32) * state_ref[0] + base_ref[...]
    o_ref[0] = jax.lax.dot_general(
        x_ref[0], w, (((1,), (1,)), ((), ())),
        preferred_element_type=jnp.float32)


def _call(x, log_lr, state, base_w, interpret=False):
    n_ob = _DOUT // _BO
    return pl.pallas_call(
        _body,
        out_shape=jax.ShapeDtypeStruct((_B, _S, _DOUT), jnp.float32),
        grid=(n_ob, _B),
        in_specs=[
            pl.BlockSpec((1, _S, _DIN), lambda ob, b: (b, 0, 0)),
            pl.BlockSpec((_BO, _DIN), lambda ob, b: (ob, 0)),
            pl.BlockSpec((1, _BO, _DIN), lambda ob, b: (b, ob, 0)),
            pl.BlockSpec((_BO, _DIN), lambda ob, b: (ob, 0)),
        ],
        out_specs=pl.BlockSpec((1, _S, _BO), lambda ob, b: (b, 0, ob)),
        scratch_shapes=[pltpu.VMEM((_BO, _DIN), jnp.bfloat16)],
        compiler_params=pltpu.CompilerParams(
            dimension_semantics=("parallel", "arbitrary"),
            vmem_limit_bytes=56 * 1024 * 1024,
        ),
        name="ittt_linear",
        interpret=interpret,
    )(x, log_lr, state, base_w)


def kernel(x, log_lr, state, momentum, base_w):
    del momentum
    return _call(x, log_lr, state, base_w)
